# Initial kernel scaffold; baseline (speedup 1.0000x reference)
import functools

import jax
import jax.numpy as jnp
from jax import lax
from jax.experimental import pallas as pl
from jax.experimental.pallas import tpu as pltpu
from jax.experimental.pallas import tpu_sc as plsc

NC = 2
NS = 16
NW = NC * NS
LANES = 16

_sc_mesh = functools.partial(
    plsc.VectorSubcoreMesh,
    core_axis_name="c", subcore_axis_name="s", num_cores=NC, num_subcores=NS,
)


def _worker_id():
    return lax.axis_index("s") * NC + lax.axis_index("c")


def _make_deg_kernel(N, E):
    epw = E // NW

    @functools.partial(
        pl.kernel,
        out_type=jax.ShapeDtypeStruct((NW, N), jnp.float32),
        mesh=_sc_mesh(),
        scratch_types=[
            pltpu.VMEM((epw,), jnp.int32),
            pltpu.VMEM((N,), jnp.float32),
        ],
    )
    def deg_kernel(dst_hbm, out_hbm, idx_v, hist_v):
        w = _worker_id()
        pltpu.sync_copy(dst_hbm.at[pl.ds(w * epw, epw)], idx_v)

        def zero_body(i, _):
            hist_v[pl.ds(i * LANES, LANES)] = jnp.zeros((LANES,), jnp.float32)
            return _

        lax.fori_loop(0, N // LANES, zero_body, None)

        ones = jnp.ones((LANES,), jnp.float32)

        def body(j, _):
            idx = idx_v[pl.ds(j * LANES, LANES)]
            plsc.addupdate_scatter(hist_v, [idx], ones)
            return _

        lax.fori_loop(0, epw // LANES, body, None)
        pltpu.sync_copy(hist_v, out_hbm.at[w])

    return deg_kernel


def _make_agg_kernel(N, E, F):
    epw = E // NW
    K = 80
    nit = epw // K
    rpt = N // NS
    zrows = 125

    @functools.partial(
        pl.kernel,
        out_type=jax.ShapeDtypeStruct((NC, N, F), jnp.float32),
        mesh=_sc_mesh(),
        scratch_types=[
            pltpu.VMEM((K,), jnp.int32),
            pltpu.VMEM((K,), jnp.int32),
            pltpu.VMEM((K, F), jnp.float32),
            pltpu.VMEM((zrows, F), jnp.float32),
            pltpu.VMEM_SHARED((N, F), jnp.float32),
            pltpu.SemaphoreType.DMA,
        ],
    )
    def agg_kernel(h_hbm, src_hbm, dst_hbm, out_hbm, sidx, didx, rows, zbuf,
                   acc, sem):
        c = lax.axis_index("c")
        s = lax.axis_index("s")
        w = s * NC + c

        def zb_body(i, _):
            def zl_body(j, _):
                zbuf[i, pl.ds(j * LANES, LANES)] = jnp.zeros((LANES,),
                                                             jnp.float32)
                return _
            return lax.fori_loop(0, F // LANES, zl_body, _)

        lax.fori_loop(0, zrows, zb_body, None)

        def zc_body(r, _):
            pltpu.sync_copy(zbuf, acc.at[pl.ds(s * rpt + r * zrows, zrows)])
            return _

        lax.fori_loop(0, rpt // zrows, zc_body, None)
        plsc.subcore_barrier()

        def body(j, _):
            base = w * epw + j * K
            pltpu.sync_copy(src_hbm.at[pl.ds(base, K)], sidx)
            pltpu.sync_copy(dst_hbm.at[pl.ds(base, K)], didx)
            pltpu.async_copy(h_hbm.at[sidx], rows, sem).wait()
            pltpu.sync_copy(rows, acc.at[didx], add=True)
            return _

        lax.fori_loop(0, nit, body, None)
        plsc.subcore_barrier()

        def out_body(r, _):
            ro = s * rpt + r * zrows
            pltpu.sync_copy(acc.at[pl.ds(ro, zrows)],
                            out_hbm.at[c, pl.ds(ro, zrows)])
            return _

        lax.fori_loop(0, rpt // zrows, out_body, None)

    return agg_kernel


def _dinv_from(degs_t):
    deg = jnp.sum(degs_t, axis=1, keepdims=True)
    return jnp.where(deg > 0, lax.rsqrt(jnp.maximum(deg, 1e-12)), 0.0)


def _mm1_body(x_ref, w_ref, degs_ref, out_ref):
    dinv = _dinv_from(degs_ref[...])
    h = jnp.dot(x_ref[...], w_ref[...], preferred_element_type=jnp.float32)
    out_ref[...] = h * dinv


def _mid_body(p_ref, degs_ref, b_ref, w_ref, out_ref):
    dinv = _dinv_from(degs_ref[...])
    a = (p_ref[0] + p_ref[1]) * dinv + b_ref[...]
    z = jnp.maximum(a, 0.0)
    h = jnp.dot(z, w_ref[...], preferred_element_type=jnp.float32)
    out_ref[...] = h * dinv


def _out_body(p_ref, degs_ref, b_ref, out_ref):
    dinv = _dinv_from(degs_ref[...])
    o = (p_ref[0] + p_ref[1]) * dinv + b_ref[...]
    m = jnp.max(o, axis=1, keepdims=True)
    e = jnp.exp(o - m)
    ssum = jnp.sum(e, axis=1, keepdims=True)
    out_ref[...] = (o - m) - jnp.log(ssum)


def kernel(x, edge_index, W1, b1, W2, b2):
    N, F_IN = x.shape
    H = W1.shape[1]
    C = W2.shape[1]
    E = edge_index.shape[1]
    src = edge_index[0]
    dst = edge_index[1]

    degs = _make_deg_kernel(N, E)(dst)
    degs_t = degs.T

    BN = 1000
    grid = (N // BN,)
    row_spec = lambda F: pl.BlockSpec((BN, F), lambda i: (i, 0))
    degs_spec = pl.BlockSpec((BN, NW), lambda i: (i, 0))
    part_spec = lambda F: pl.BlockSpec((NC, BN, F), lambda i: (0, i, 0))
    full = lambda a, b: pl.BlockSpec((a, b), lambda i: (0, 0))

    h1s = pl.pallas_call(
        _mm1_body,
        grid=grid,
        in_specs=[row_spec(F_IN), full(F_IN, H), degs_spec],
        out_specs=row_spec(H),
        out_shape=jax.ShapeDtypeStruct((N, H), jnp.float32),
    )(x, W1, degs_t)

    p1 = _make_agg_kernel(N, E, H)(h1s, src, dst)

    h2s = pl.pallas_call(
        _mid_body,
        grid=grid,
        in_specs=[part_spec(H), degs_spec, full(1, H), full(H, C)],
        out_specs=row_spec(C),
        out_shape=jax.ShapeDtypeStruct((N, C), jnp.float32),
    )(p1, degs_t, b1.reshape(1, H), W2)

    p2 = _make_agg_kernel(N, E, C)(h2s, src, dst)

    out = pl.pallas_call(
        _out_body,
        grid=grid,
        in_specs=[part_spec(C), degs_spec, full(1, C)],
        out_specs=row_spec(C),
        out_shape=jax.ShapeDtypeStruct((N, C), jnp.float32),
    )(p2, degs_t, b2.reshape(1, C))

    return out


# speedup vs baseline: 11.3774x; 11.3774x over previous
import functools

import jax
import jax.numpy as jnp
from jax import lax
from jax.experimental import pallas as pl
from jax.experimental.pallas import tpu as pltpu
from jax.experimental.pallas import tpu_sc as plsc

NC = 2
NS = 16
NW = NC * NS
LANES = 16

_sc_mesh = functools.partial(
    plsc.VectorSubcoreMesh,
    core_axis_name="c", subcore_axis_name="s", num_cores=NC, num_subcores=NS,
)


def _worker_id():
    return lax.axis_index("s") * NC + lax.axis_index("c")


def _make_deg_kernel(N, E):
    epw = E // NW
    K = 80
    nit = epw // K
    FD = 16
    Z = 200
    nch = N // Z
    rmax = (nch + NS - 1) // NS

    @functools.partial(
        pl.kernel,
        out_type=jax.ShapeDtypeStruct((NC, N, FD), jnp.float32),
        mesh=_sc_mesh(),
        compiler_params=pltpu.CompilerParams(use_tc_tiling_on_sc=False),
        scratch_types=[
            pltpu.VMEM((K,), jnp.int32),
            pltpu.VMEM((K, FD), jnp.float32),
            pltpu.VMEM((Z, FD), jnp.float32),
            pltpu.VMEM_SHARED((N, FD), jnp.float32),
        ],
    )
    def deg_kernel(dst_hbm, out_hbm, didx, ones_v, zbuf, acc):
        c = lax.axis_index("c")
        s = lax.axis_index("s")
        w = s * NC + c

        def fill_body(i, _):
            ones_v[i, :] = jnp.ones((FD,), jnp.float32)
            return _

        lax.fori_loop(0, K, fill_body, None)

        def zb_body(i, _):
            zbuf[i, :] = jnp.zeros((FD,), jnp.float32)
            return _

        lax.fori_loop(0, Z, zb_body, None)

        def zc_body(r, _):
            m = s + NS * r

            @pl.when(m < nch)
            def _do():
                pltpu.sync_copy(zbuf, acc.at[pl.ds(m * Z, Z)])

            return _

        lax.fori_loop(0, rmax, zc_body, None)
        plsc.subcore_barrier()

        def body(j, _):
            pltpu.sync_copy(dst_hbm.at[pl.ds(w * epw + j * K, K)], didx)
            pltpu.sync_copy(ones_v, acc.at[didx], add=True)
            return _

        lax.fori_loop(0, nit, body, None)
        plsc.subcore_barrier()

        def out_body(r, _):
            m = s + NS * r

            @pl.when(m < nch)
            def _do():
                pltpu.sync_copy(acc.at[pl.ds(m * Z, Z)],
                                out_hbm.at[c, pl.ds(m * Z, Z)])

            return _

        lax.fori_loop(0, rmax, out_body, None)

    return deg_kernel


def _make_agg_kernel(N, E, F):
    epw = E // NW
    K = 80
    nit = epw // K
    Z = 200
    nch = N // Z
    rmax = (nch + NS - 1) // NS

    @functools.partial(
        pl.kernel,
        out_type=jax.ShapeDtypeStruct((NC, N, F), jnp.float32),
        mesh=_sc_mesh(),
        compiler_params=pltpu.CompilerParams(use_tc_tiling_on_sc=False),
        scratch_types=[
            pltpu.VMEM((K,), jnp.int32),
            pltpu.VMEM((K,), jnp.int32),
            pltpu.VMEM((K, F), jnp.float32),
            pltpu.VMEM((Z, F), jnp.float32),
            pltpu.VMEM_SHARED((N, F), jnp.float32),
            pltpu.SemaphoreType.DMA,
        ],
    )
    def agg_kernel(h_hbm, src_hbm, dst_hbm, out_hbm, sidx, didx, rows, zbuf,
                   acc, sem):
        c = lax.axis_index("c")
        s = lax.axis_index("s")
        w = s * NC + c

        def zb_body(i, _):
            def zl_body(j, _):
                zbuf[i, pl.ds(j * LANES, LANES)] = jnp.zeros((LANES,),
                                                             jnp.float32)
                return _
            return lax.fori_loop(0, F // LANES, zl_body, _)

        lax.fori_loop(0, Z, zb_body, None)

        def zc_body(r, _):
            m = s + NS * r

            @pl.when(m < nch)
            def _do():
                pltpu.sync_copy(zbuf, acc.at[pl.ds(m * Z, Z)])

            return _

        lax.fori_loop(0, rmax, zc_body, None)
        plsc.subcore_barrier()

        def body(j, _):
            base = w * epw + j * K
            pltpu.sync_copy(src_hbm.at[pl.ds(base, K)], sidx)
            pltpu.sync_copy(dst_hbm.at[pl.ds(base, K)], didx)
            pltpu.async_copy(h_hbm.at[sidx], rows, sem).wait()
            pltpu.sync_copy(rows, acc.at[didx], add=True)
            return _

        lax.fori_loop(0, nit, body, None)
        plsc.subcore_barrier()

        def out_body(r, _):
            m = s + NS * r

            @pl.when(m < nch)
            def _do():
                pltpu.sync_copy(acc.at[pl.ds(m * Z, Z)],
                                out_hbm.at[c, pl.ds(m * Z, Z)])

            return _

        lax.fori_loop(0, rmax, out_body, None)

    return agg_kernel


def _dinv_from(degs_ref):
    deg = degs_ref[0, :, 0:1] + degs_ref[1, :, 0:1]
    return jnp.where(deg > 0, lax.rsqrt(jnp.maximum(deg, 1e-12)), 0.0)


def _mm1_body(x_ref, w_ref, degs_ref, out_ref):
    dinv = _dinv_from(degs_ref)
    h = jnp.dot(x_ref[...], w_ref[...], preferred_element_type=jnp.float32)
    out_ref[...] = h * dinv


def _mid_body(p_ref, degs_ref, b_ref, w_ref, out_ref):
    dinv = _dinv_from(degs_ref)
    a = (p_ref[0] + p_ref[1]) * dinv + b_ref[...]
    z = jnp.maximum(a, 0.0)
    h = jnp.dot(z, w_ref[...], preferred_element_type=jnp.float32)
    out_ref[...] = h * dinv


def _out_body(p_ref, degs_ref, b_ref, out_ref):
    dinv = _dinv_from(degs_ref)
    o = (p_ref[0] + p_ref[1]) * dinv + b_ref[...]
    m = jnp.max(o, axis=1, keepdims=True)
    e = jnp.exp(o - m)
    ssum = jnp.sum(e, axis=1, keepdims=True)
    out_ref[...] = (o - m) - jnp.log(ssum)


def kernel(x, edge_index, W1, b1, W2, b2):
    N, F_IN = x.shape
    H = W1.shape[1]
    C = W2.shape[1]
    E = edge_index.shape[1]
    src = edge_index[0]
    dst = edge_index[1]

    degs = _make_deg_kernel(N, E)(dst)

    BN = 1000
    grid = (N // BN,)
    row_spec = lambda F: pl.BlockSpec((BN, F), lambda i: (i, 0))
    degs_spec = pl.BlockSpec((NC, BN, 16), lambda i: (0, i, 0))
    part_spec = lambda F: pl.BlockSpec((NC, BN, F), lambda i: (0, i, 0))
    full = lambda a, b: pl.BlockSpec((a, b), lambda i: (0, 0))

    h1s = pl.pallas_call(
        _mm1_body,
        grid=grid,
        in_specs=[row_spec(F_IN), full(F_IN, H), degs_spec],
        out_specs=row_spec(H),
        out_shape=jax.ShapeDtypeStruct((N, H), jnp.float32),
    )(x, W1, degs)

    p1 = _make_agg_kernel(N, E, H)(h1s, src, dst)

    h2s = pl.pallas_call(
        _mid_body,
        grid=grid,
        in_specs=[part_spec(H), degs_spec, full(1, H), full(H, C)],
        out_specs=row_spec(C),
        out_shape=jax.ShapeDtypeStruct((N, C), jnp.float32),
    )(p1, degs, b1.reshape(1, H), W2)

    p2 = _make_agg_kernel(N, E, C)(h2s, src, dst)

    out = pl.pallas_call(
        _out_body,
        grid=grid,
        in_specs=[part_spec(C), degs_spec, full(1, C)],
        out_specs=row_spec(C),
        out_shape=jax.ShapeDtypeStruct((N, C), jnp.float32),
    )(p2, degs, b2.reshape(1, C))

    return out


# speedup vs baseline: 22.8057x; 2.0045x over previous
import functools

import jax
import jax.numpy as jnp
from jax import lax
from jax.experimental import pallas as pl
from jax.experimental.pallas import tpu as pltpu
from jax.experimental.pallas import tpu_sc as plsc

NC = 2
NS = 16
NW = NC * NS
LANES = 16

_sc_mesh = functools.partial(
    plsc.VectorSubcoreMesh,
    core_axis_name="c", subcore_axis_name="s", num_cores=NC, num_subcores=NS,
)


def _worker_id():
    return lax.axis_index("s") * NC + lax.axis_index("c")


def _make_deg_kernel(N, E):
    epw = E // NW
    K = 80
    nit = epw // K
    FD = 16
    Z = 200
    nch = N // Z
    rmax = (nch + NS - 1) // NS

    @functools.partial(
        pl.kernel,
        out_type=jax.ShapeDtypeStruct((NC, N, FD), jnp.float32),
        mesh=_sc_mesh(),
        compiler_params=pltpu.CompilerParams(use_tc_tiling_on_sc=False),
        scratch_types=[
            pltpu.VMEM((K,), jnp.int32),
            pltpu.VMEM((K, FD), jnp.float32),
            pltpu.VMEM((Z, FD), jnp.float32),
            pltpu.VMEM_SHARED((N, FD), jnp.float32),
        ],
    )
    def deg_kernel(dst_hbm, out_hbm, didx, ones_v, zbuf, acc):
        c = lax.axis_index("c")
        s = lax.axis_index("s")
        w = s * NC + c

        def fill_body(i, _):
            ones_v[i, :] = jnp.ones((FD,), jnp.float32)
            return _

        lax.fori_loop(0, K, fill_body, None)

        def zb_body(i, _):
            zbuf[i, :] = jnp.zeros((FD,), jnp.float32)
            return _

        lax.fori_loop(0, Z, zb_body, None)

        def zc_body(r, _):
            m = s + NS * r

            @pl.when(m < nch)
            def _do():
                pltpu.sync_copy(zbuf, acc.at[pl.ds(m * Z, Z)])

            return _

        lax.fori_loop(0, rmax, zc_body, None)
        plsc.subcore_barrier()

        def body(j, _):
            pltpu.sync_copy(dst_hbm.at[pl.ds(w * epw + j * K, K)], didx)
            pltpu.sync_copy(ones_v, acc.at[didx], add=True)
            return _

        lax.fori_loop(0, nit, body, None)
        plsc.subcore_barrier()

        def out_body(r, _):
            m = s + NS * r

            @pl.when(m < nch)
            def _do():
                pltpu.sync_copy(acc.at[pl.ds(m * Z, Z)],
                                out_hbm.at[c, pl.ds(m * Z, Z)])

            return _

        lax.fori_loop(0, rmax, out_body, None)

    return deg_kernel


AGG_K = 100


def _make_agg_kernel(N, E, F):
    epw = E // NW
    K = AGG_K
    nit = epw // K
    Z = 200
    nch = N // Z
    rmax = (nch + NS - 1) // NS

    @functools.partial(
        pl.kernel,
        out_type=jax.ShapeDtypeStruct((NC, N, F), jnp.float32),
        mesh=_sc_mesh(),
        compiler_params=pltpu.CompilerParams(use_tc_tiling_on_sc=False),
        scratch_types=[
            pltpu.VMEM((nit, K), jnp.int32),
            pltpu.VMEM((nit, K), jnp.int32),
            pltpu.VMEM((2, K, F), jnp.float32),
            pltpu.VMEM_SHARED((N, F), jnp.float32),
            pltpu.SemaphoreType.DMA,
            pltpu.SemaphoreType.DMA,
            pltpu.SemaphoreType.DMA,
            pltpu.SemaphoreType.DMA,
            pltpu.SemaphoreType.DMA,
        ],
    )
    def agg_kernel(h_hbm, src_hbm, dst_hbm, z_hbm, out_hbm, sidx, didx, rows,
                   acc, sg0, sg1, ss0, ss1, sz):
        c = lax.axis_index("c")
        s = lax.axis_index("s")
        w = s * NC + c
        SG = (sg0, sg1)
        SS = (ss0, ss1)

        def zf_body(r, _):
            m = s + NS * r

            @pl.when(m < nch)
            def _do():
                pltpu.async_copy(z_hbm, acc.at[pl.ds(m * Z, Z)], sz)

            return _

        lax.fori_loop(0, rmax, zf_body, None)

        pltpu.sync_copy(src_hbm.at[pl.ds(w * nit, nit)], sidx)
        pltpu.sync_copy(dst_hbm.at[pl.ds(w * nit, nit)], didx)

        def zd_body(r, _):
            m = s + NS * r

            @pl.when(m < nch)
            def _do():
                pltpu.make_async_copy(z_hbm, acc.at[pl.ds(m * Z, Z)],
                                      sz).wait()

            return _

        lax.fori_loop(0, rmax, zd_body, None)
        plsc.subcore_barrier()

        pltpu.async_copy(h_hbm.at[sidx.at[0]], rows.at[0], sg0)

        def body(g, _):
            for b in range(2):
                j = g * 2 + b
                nb = 1 - b

                @pl.when(j + 1 < nit)
                def _prefetch():
                    @pl.when(j >= 1)
                    def _drain():
                        pltpu.make_async_copy(
                            rows.at[nb], acc.at[didx.at[j - 1]],
                            SS[nb]).wait()

                    pltpu.async_copy(h_hbm.at[sidx.at[j + 1]], rows.at[nb],
                                     SG[nb])

                pltpu.make_async_copy(h_hbm.at[sidx.at[j]], rows.at[b],
                                      SG[b]).wait()
                pltpu.async_copy(rows.at[b], acc.at[didx.at[j]], SS[b],
                                 add=True)
            return _

        lax.fori_loop(0, nit // 2, body, None)
        for b in range(2):
            pltpu.make_async_copy(rows.at[b], acc.at[didx.at[b]],
                                  SS[b]).wait()
        plsc.subcore_barrier()

        def out_body(r, _):
            m = s + NS * r

            @pl.when(m < nch)
            def _do():
                pltpu.sync_copy(acc.at[pl.ds(m * Z, Z)],
                                out_hbm.at[c, pl.ds(m * Z, Z)])

            return _

        lax.fori_loop(0, rmax, out_body, None)

    return agg_kernel


def _dinv_from(degs_ref):
    deg = degs_ref[0, :, 0:1] + degs_ref[1, :, 0:1]
    return jnp.where(deg > 0, lax.rsqrt(jnp.maximum(deg, 1e-12)), 0.0)


def _mm1_body(x_ref, w_ref, degs_ref, out_ref):
    dinv = _dinv_from(degs_ref)
    h = jnp.dot(x_ref[...], w_ref[...], preferred_element_type=jnp.float32)
    out_ref[...] = h * dinv


def _mid_body(p_ref, degs_ref, b_ref, w_ref, out_ref):
    dinv = _dinv_from(degs_ref)
    a = (p_ref[0] + p_ref[1]) * dinv + b_ref[...]
    z = jnp.maximum(a, 0.0)
    h = jnp.dot(z, w_ref[...], preferred_element_type=jnp.float32)
    out_ref[...] = h * dinv


def _out_body(p_ref, degs_ref, b_ref, out_ref):
    dinv = _dinv_from(degs_ref)
    o = (p_ref[0] + p_ref[1]) * dinv + b_ref[...]
    m = jnp.max(o, axis=1, keepdims=True)
    e = jnp.exp(o - m)
    ssum = jnp.sum(e, axis=1, keepdims=True)
    out_ref[...] = (o - m) - jnp.log(ssum)


def kernel(x, edge_index, W1, b1, W2, b2):
    N, F_IN = x.shape
    H = W1.shape[1]
    C = W2.shape[1]
    E = edge_index.shape[1]
    src = edge_index[0]
    dst = edge_index[1]

    degs = _make_deg_kernel(N, E)(dst)
    src2 = src.reshape(E // AGG_K, AGG_K)
    dst2 = dst.reshape(E // AGG_K, AGG_K)

    BN = 1000
    grid = (N // BN,)
    row_spec = lambda F: pl.BlockSpec((BN, F), lambda i: (i, 0))
    degs_spec = pl.BlockSpec((NC, BN, 16), lambda i: (0, i, 0))
    part_spec = lambda F: pl.BlockSpec((NC, BN, F), lambda i: (0, i, 0))
    full = lambda a, b: pl.BlockSpec((a, b), lambda i: (0, 0))

    h1s = pl.pallas_call(
        _mm1_body,
        grid=grid,
        in_specs=[row_spec(F_IN), full(F_IN, H), degs_spec],
        out_specs=row_spec(H),
        out_shape=jax.ShapeDtypeStruct((N, H), jnp.float32),
    )(x, W1, degs)

    p1 = _make_agg_kernel(N, E, H)(h1s, src2, dst2, jnp.zeros((200, H), jnp.float32))

    h2s = pl.pallas_call(
        _mid_body,
        grid=grid,
        in_specs=[part_spec(H), degs_spec, full(1, H), full(H, C)],
        out_specs=row_spec(C),
        out_shape=jax.ShapeDtypeStruct((N, C), jnp.float32),
    )(p1, degs, b1.reshape(1, H), W2)

    p2 = _make_agg_kernel(N, E, C)(h2s, src2, dst2, jnp.zeros((200, C), jnp.float32))

    out = pl.pallas_call(
        _out_body,
        grid=grid,
        in_specs=[part_spec(C), degs_spec, full(1, C)],
        out_specs=row_spec(C),
        out_shape=jax.ShapeDtypeStruct((N, C), jnp.float32),
    )(p2, degs, b2.reshape(1, C))

    return out


# speedup vs baseline: 26.5423x; 1.1638x over previous
import functools

import jax
import jax.numpy as jnp
from jax import lax
from jax.experimental import pallas as pl
from jax.experimental.pallas import tpu as pltpu
from jax.experimental.pallas import tpu_sc as plsc

NC = 2
NS = 16
NW = NC * NS
LANES = 16

_sc_mesh = functools.partial(
    plsc.VectorSubcoreMesh,
    core_axis_name="c", subcore_axis_name="s", num_cores=NC, num_subcores=NS,
)


def _worker_id():
    return lax.axis_index("s") * NC + lax.axis_index("c")


DEG_DEPTH = 4


def _make_deg_kernel(N, E):
    K = AGG_K
    epw = E // NW
    nit = epw // K
    FD = 16
    Z = 200
    nch = N // Z
    rmax = (nch + NS - 1) // NS

    @functools.partial(
        pl.kernel,
        out_type=jax.ShapeDtypeStruct((NC, N, FD), jnp.float32),
        mesh=_sc_mesh(),
        compiler_params=pltpu.CompilerParams(use_tc_tiling_on_sc=False),
        scratch_types=[
            pltpu.VMEM((nit, K), jnp.int32),
            pltpu.VMEM((K, FD), jnp.float32),
            pltpu.VMEM_SHARED((N, FD), jnp.float32),
            pltpu.SemaphoreType.DMA,
            pltpu.SemaphoreType.DMA,
        ],
    )
    def deg_kernel(dst_hbm, z_hbm, out_hbm, didx, ones_v, acc, ss, sz):
        c = lax.axis_index("c")
        s = lax.axis_index("s")
        w = s * NC + c

        def zf_body(r, _):
            m = s + NS * r

            @pl.when(m < nch)
            def _do():
                pltpu.async_copy(z_hbm, acc.at[pl.ds(m * Z, Z)], sz)

            return _

        lax.fori_loop(0, rmax, zf_body, None)

        pltpu.sync_copy(dst_hbm.at[pl.ds(w * nit, nit)], didx)

        def fill_body(i, _):
            ones_v[i, :] = jnp.ones((FD,), jnp.float32)
            return _

        lax.fori_loop(0, K, fill_body, None)

        def zd_body(r, _):
            m = s + NS * r

            @pl.when(m < nch)
            def _do():
                pltpu.make_async_copy(z_hbm, acc.at[pl.ds(m * Z, Z)],
                                      sz).wait()

            return _

        lax.fori_loop(0, rmax, zd_body, None)
        plsc.subcore_barrier()

        def body(j, _):
            pltpu.async_copy(ones_v, acc.at[didx.at[j]], ss, add=True)

            @pl.when(j >= DEG_DEPTH)
            def _drain():
                pltpu.make_async_copy(ones_v, acc.at[didx.at[0]], ss).wait()

            return _

        lax.fori_loop(0, nit, body, None)
        for _ in range(DEG_DEPTH):
            pltpu.make_async_copy(ones_v, acc.at[didx.at[0]], ss).wait()
        plsc.subcore_barrier()

        def out_body(r, _):
            m = s + NS * r

            @pl.when(m < nch)
            def _do():
                pltpu.sync_copy(acc.at[pl.ds(m * Z, Z)],
                                out_hbm.at[c, pl.ds(m * Z, Z)])

            return _

        lax.fori_loop(0, rmax, out_body, None)

    return deg_kernel


AGG_K = 100


def _make_agg_kernel(N, E, F):
    epw = E // NW
    K = AGG_K
    nit = epw // K
    Z = 200
    nch = N // Z
    rmax = (nch + NS - 1) // NS

    @functools.partial(
        pl.kernel,
        out_type=jax.ShapeDtypeStruct((NC, N, F), jnp.float32),
        mesh=_sc_mesh(),
        compiler_params=pltpu.CompilerParams(use_tc_tiling_on_sc=False),
        scratch_types=[
            pltpu.VMEM((nit, K), jnp.int32),
            pltpu.VMEM((nit, K), jnp.int32),
            pltpu.VMEM((2, K, F), jnp.float32),
            pltpu.VMEM_SHARED((N, F), jnp.float32),
            pltpu.SemaphoreType.DMA,
            pltpu.SemaphoreType.DMA,
            pltpu.SemaphoreType.DMA,
            pltpu.SemaphoreType.DMA,
            pltpu.SemaphoreType.DMA,
        ],
    )
    def agg_kernel(h_hbm, src_hbm, dst_hbm, z_hbm, out_hbm, sidx, didx, rows,
                   acc, sg0, sg1, ss0, ss1, sz):
        c = lax.axis_index("c")
        s = lax.axis_index("s")
        w = s * NC + c
        SG = (sg0, sg1)
        SS = (ss0, ss1)

        def zf_body(r, _):
            m = s + NS * r

            @pl.when(m < nch)
            def _do():
                pltpu.async_copy(z_hbm, acc.at[pl.ds(m * Z, Z)], sz)

            return _

        lax.fori_loop(0, rmax, zf_body, None)

        pltpu.sync_copy(src_hbm.at[pl.ds(w * nit, nit)], sidx)
        pltpu.sync_copy(dst_hbm.at[pl.ds(w * nit, nit)], didx)

        def zd_body(r, _):
            m = s + NS * r

            @pl.when(m < nch)
            def _do():
                pltpu.make_async_copy(z_hbm, acc.at[pl.ds(m * Z, Z)],
                                      sz).wait()

            return _

        lax.fori_loop(0, rmax, zd_body, None)
        plsc.subcore_barrier()

        pltpu.async_copy(h_hbm.at[sidx.at[0]], rows.at[0], sg0)

        def body(g, _):
            for b in range(2):
                j = g * 2 + b
                nb = 1 - b

                @pl.when(j + 1 < nit)
                def _prefetch():
                    @pl.when(j >= 1)
                    def _drain():
                        pltpu.make_async_copy(
                            rows.at[nb], acc.at[didx.at[j - 1]],
                            SS[nb]).wait()

                    pltpu.async_copy(h_hbm.at[sidx.at[j + 1]], rows.at[nb],
                                     SG[nb])

                pltpu.make_async_copy(h_hbm.at[sidx.at[j]], rows.at[b],
                                      SG[b]).wait()
                pltpu.async_copy(rows.at[b], acc.at[didx.at[j]], SS[b],
                                 add=True)
            return _

        lax.fori_loop(0, nit // 2, body, None)
        for b in range(2):
            pltpu.make_async_copy(rows.at[b], acc.at[didx.at[b]],
                                  SS[b]).wait()
        plsc.subcore_barrier()

        def out_body(r, _):
            m = s + NS * r

            @pl.when(m < nch)
            def _do():
                pltpu.sync_copy(acc.at[pl.ds(m * Z, Z)],
                                out_hbm.at[c, pl.ds(m * Z, Z)])

            return _

        lax.fori_loop(0, rmax, out_body, None)

    return agg_kernel


def _dinv_from(degs_ref):
    deg = degs_ref[0, :, 0:1] + degs_ref[1, :, 0:1]
    return jnp.where(deg > 0, lax.rsqrt(jnp.maximum(deg, 1e-12)), 0.0)


def _mm1_body(x_ref, w_ref, degs_ref, out_ref):
    dinv = _dinv_from(degs_ref)
    h = jnp.dot(x_ref[...], w_ref[...], preferred_element_type=jnp.float32)
    out_ref[...] = h * dinv


def _mid_body(p_ref, degs_ref, b_ref, w_ref, out_ref):
    dinv = _dinv_from(degs_ref)
    a = (p_ref[0] + p_ref[1]) * dinv + b_ref[...]
    z = jnp.maximum(a, 0.0)
    h = jnp.dot(z, w_ref[...], preferred_element_type=jnp.float32)
    out_ref[...] = h * dinv


def _out_body(p_ref, degs_ref, b_ref, out_ref):
    dinv = _dinv_from(degs_ref)
    o = (p_ref[0] + p_ref[1]) * dinv + b_ref[...]
    m = jnp.max(o, axis=1, keepdims=True)
    e = jnp.exp(o - m)
    ssum = jnp.sum(e, axis=1, keepdims=True)
    out_ref[...] = (o - m) - jnp.log(ssum)


def kernel(x, edge_index, W1, b1, W2, b2):
    N, F_IN = x.shape
    H = W1.shape[1]
    C = W2.shape[1]
    E = edge_index.shape[1]
    src = edge_index[0]
    dst = edge_index[1]

    src2 = src.reshape(E // AGG_K, AGG_K)
    dst2 = dst.reshape(E // AGG_K, AGG_K)
    degs = _make_deg_kernel(N, E)(dst2, jnp.zeros((200, 16), jnp.float32))

    BN = 1000
    grid = (N // BN,)
    row_spec = lambda F: pl.BlockSpec((BN, F), lambda i: (i, 0))
    degs_spec = pl.BlockSpec((NC, BN, 16), lambda i: (0, i, 0))
    part_spec = lambda F: pl.BlockSpec((NC, BN, F), lambda i: (0, i, 0))
    full = lambda a, b: pl.BlockSpec((a, b), lambda i: (0, 0))

    h1s = pl.pallas_call(
        _mm1_body,
        grid=grid,
        in_specs=[row_spec(F_IN), full(F_IN, H), degs_spec],
        out_specs=row_spec(H),
        out_shape=jax.ShapeDtypeStruct((N, H), jnp.float32),
    )(x, W1, degs)

    p1 = _make_agg_kernel(N, E, H)(h1s, src2, dst2, jnp.zeros((200, H), jnp.float32))

    h2s = pl.pallas_call(
        _mid_body,
        grid=grid,
        in_specs=[part_spec(H), degs_spec, full(1, H), full(H, C)],
        out_specs=row_spec(C),
        out_shape=jax.ShapeDtypeStruct((N, C), jnp.float32),
    )(p1, degs, b1.reshape(1, H), W2)

    p2 = _make_agg_kernel(N, E, C)(h2s, src2, dst2, jnp.zeros((200, C), jnp.float32))

    out = pl.pallas_call(
        _out_body,
        grid=grid,
        in_specs=[part_spec(C), degs_spec, full(1, C)],
        out_specs=row_spec(C),
        out_shape=jax.ShapeDtypeStruct((N, C), jnp.float32),
    )(p2, degs, b2.reshape(1, C))

    return out
